# baseline (speedup 1.0000x reference)
import functools

import jax
import jax.numpy as jnp
from jax.experimental import pallas as pl

N_FEAT = 32
N_BINS = 64
N_PAR = 3 * N_BINS + 1
BLK = 8
CH = 128
NC = N_BINS * N_FEAT // CH


def _spline_block_kernel(x_ref, p_ref, x0_ref, xf_ref, y0_ref, yf_ref,
                         y_ref, ld_ref):
    f32 = jnp.float32
    x = x_ref[...]
    x0 = x0_ref[...]
    xf = xf_ref[...]
    y0 = y0_ref[...]
    yf = yf_ref[...]
    blk = x.shape[0]

    li = jax.lax.broadcasted_iota(jnp.int32, (1, CH), 1)
    m_ge32 = li >= 32
    m_ge64 = li >= 64
    m_ge96 = li >= 96
    m_lt32 = li < 32
    m_lt96 = li < 96

    def roll(v, n):
        return pltpu_roll(v, n)

    carry = jnp.zeros((blk, CH), f32)
    cums = []
    for kc in range(NC):
        ew = jnp.exp(p_ref[:, kc * CH:(kc + 1) * CH])
        a1 = ew + jnp.where(m_ge32, roll(ew, 32), 0.0)
        a2 = a1 + jnp.where(m_ge64, roll(a1, 64), 0.0)
        cums.append(carry + a2)
        tot = jnp.where(m_ge96, a2, 0.0)
        tot = tot + roll(tot, 32)
        tot = tot + roll(tot, 64)
        carry = carry + tot

    sw32 = carry[:, :N_FEAT]
    tb32 = (x - x0) * sw32 / (xf - x0)
    tb = jnp.concatenate([tb32, tb32, tb32, tb32], axis=1)

    zero = jnp.zeros((blk, CH), f32)
    prev_cum = zero
    prev_eq = zero
    xk_acc = zero
    we_acc = zero
    sh_acc = zero
    yk_acc = zero
    h_acc = zero
    dk_acc = zero
    dk1_acc = zero
    off_h = N_BINS * N_FEAT
    off_s = 2 * N_BINS * N_FEAT
    for kc in range(NC):
        cumc = cums[kc]
        cumprev = jnp.where(m_lt32, roll(prev_cum, 32), roll(cumc, 32))
        lt = jnp.where(tb > cumc, 1.0, 0.0)
        if kc == NC - 1:
            lt = jnp.where(m_lt96, lt, 0.0)
        ltp = jnp.where(tb > cumprev, 1.0, 0.0)
        eq = ltp - lt
        xk_acc = xk_acc + cumprev * eq
        we_acc = we_acc + cumc * eq
        eh = jnp.exp(p_ref[:, off_h + kc * CH:off_h + (kc + 1) * CH])
        sh_acc = sh_acc + eh
        yk_acc = yk_acc + eh * lt
        h_acc = h_acc + eh * eq
        ps_c = p_ref[:, off_s + kc * CH:off_s + (kc + 1) * CH]
        dk_acc = dk_acc + ps_c * eq
        sh_eq = jnp.where(m_lt32, roll(prev_eq, 32), roll(eq, 32))
        dk1_acc = dk1_acc + ps_c * sh_eq
        prev_cum = cumc
        prev_eq = eq

    def lane_reduce(v):
        v2 = v[:, :64] + v[:, 64:]
        return v2[:, :N_FEAT] + v2[:, N_FEAT:]

    xk_u = lane_reduce(xk_acc)
    we_u = lane_reduce(we_acc)
    sh32 = lane_reduce(sh_acc)
    yk_u = lane_reduce(yk_acc)
    h_u = lane_reduce(h_acc)
    dk_logit = lane_reduce(dk_acc)
    dk1_logit = lane_reduce(dk1_acc)

    eq63 = prev_eq[:, 96:]
    ps_last = p_ref[:, N_PAR * N_FEAT - N_FEAT:]
    dk1_logit = dk1_logit + ps_last * eq63

    cxn = (xf - x0) / sw32
    cyn = (yf - y0) / sh32
    w = (we_u - xk_u) * cxn
    xk = x0 + xk_u * cxn
    h = h_u * cyn
    yk = y0 + yk_u * cyn
    dk = jax.nn.softplus(dk_logit)
    dk1 = jax.nn.softplus(dk1_logit)

    s = h / w
    eps = (x - xk) / w
    e1me = eps * (1.0 - eps)
    e2 = eps * eps
    den = s + (dk1 + dk - 2.0 * s) * e1me
    y = yk + h * (s * e2 + dk * e1me) / den
    num_J = s * s * (dk1 * e2 + 2.0 * s * e1me + dk * (1.0 - eps) ** 2)
    y_ref[...] = y
    ld_ref[...] = jnp.sum(jnp.log(num_J / (den * den)), axis=1, keepdims=True)


def pltpu_roll(v, n):
    return jnp.roll(v, n, axis=1)


@functools.partial(jax.jit, static_argnames=("interpret",))
def kernel(x, parameters, x0, xf, y0, yf, interpret=False):
    batch = x.shape[0]
    p2d = parameters.reshape(batch, N_PAR * N_FEAT)
    grid = (batch // BLK,)
    y, ld = pl.pallas_call(
        _spline_block_kernel,
        grid=grid,
        in_specs=[
            pl.BlockSpec((BLK, N_FEAT), lambda i: (i, 0)),
            pl.BlockSpec((BLK, N_PAR * N_FEAT), lambda i: (i, 0)),
            pl.BlockSpec((1, N_FEAT), lambda i: (0, 0)),
            pl.BlockSpec((1, N_FEAT), lambda i: (0, 0)),
            pl.BlockSpec((1, N_FEAT), lambda i: (0, 0)),
            pl.BlockSpec((1, N_FEAT), lambda i: (0, 0)),
        ],
        out_specs=[
            pl.BlockSpec((BLK, N_FEAT), lambda i: (i, 0)),
            pl.BlockSpec((BLK, 1), lambda i: (i, 0)),
        ],
        out_shape=[
            jax.ShapeDtypeStruct((batch, N_FEAT), jnp.float32),
            jax.ShapeDtypeStruct((batch, 1), jnp.float32),
        ],
        interpret=interpret,
    )(x, p2d, x0.reshape(1, -1), xf.reshape(1, -1),
      y0.reshape(1, -1), yf.reshape(1, -1))
    return y, ld.reshape(batch)


# speedup vs baseline: 1.5210x; 1.5210x over previous
import functools

import jax
import jax.numpy as jnp
from jax.experimental import pallas as pl

N_FEAT = 32
N_BINS = 64
N_PAR = 3 * N_BINS + 1
BLK = 64
CH = 128
NC = N_BINS * N_FEAT // CH


def _spline_block_kernel(x_ref, p_ref, x0_ref, xf_ref, y0_ref, yf_ref,
                         y_ref, ld_ref):
    f32 = jnp.float32
    x = x_ref[...]
    x0 = x0_ref[...]
    xf = xf_ref[...]
    y0 = y0_ref[...]
    yf = yf_ref[...]
    blk = x.shape[0]

    li = jax.lax.broadcasted_iota(jnp.int32, (1, CH), 1)
    m_ge32 = li >= 32
    m_ge64 = li >= 64
    m_ge96 = li >= 96
    m_lt32 = li < 32
    m_lt96 = li < 96

    def roll(v, n):
        return pltpu_roll(v, n)

    carry = jnp.zeros((blk, CH), f32)
    cums = []
    for kc in range(NC):
        ew = jnp.exp(p_ref[:, kc * CH:(kc + 1) * CH])
        a1 = ew + jnp.where(m_ge32, roll(ew, 32), 0.0)
        a2 = a1 + jnp.where(m_ge64, roll(a1, 64), 0.0)
        cums.append(carry + a2)
        tot = jnp.where(m_ge96, a2, 0.0)
        tot = tot + roll(tot, 32)
        tot = tot + roll(tot, 64)
        carry = carry + tot

    sw32 = carry[:, :N_FEAT]
    tb32 = (x - x0) * sw32 / (xf - x0)
    tb = jnp.concatenate([tb32, tb32, tb32, tb32], axis=1)

    zero = jnp.zeros((blk, CH), f32)
    prev_cum = zero
    prev_eq = zero
    xk_acc = zero
    we_acc = zero
    sh_acc = zero
    yk_acc = zero
    h_acc = zero
    dk_acc = zero
    dk1_acc = zero
    off_h = N_BINS * N_FEAT
    off_s = 2 * N_BINS * N_FEAT
    for kc in range(NC):
        cumc = cums[kc]
        cumprev = jnp.where(m_lt32, roll(prev_cum, 32), roll(cumc, 32))
        lt = jnp.where(tb > cumc, 1.0, 0.0)
        if kc == NC - 1:
            lt = jnp.where(m_lt96, lt, 0.0)
        ltp = jnp.where(tb > cumprev, 1.0, 0.0)
        eq = ltp - lt
        xk_acc = xk_acc + cumprev * eq
        we_acc = we_acc + cumc * eq
        eh = jnp.exp(p_ref[:, off_h + kc * CH:off_h + (kc + 1) * CH])
        sh_acc = sh_acc + eh
        yk_acc = yk_acc + eh * lt
        h_acc = h_acc + eh * eq
        ps_c = p_ref[:, off_s + kc * CH:off_s + (kc + 1) * CH]
        dk_acc = dk_acc + ps_c * eq
        sh_eq = jnp.where(m_lt32, roll(prev_eq, 32), roll(eq, 32))
        dk1_acc = dk1_acc + ps_c * sh_eq
        prev_cum = cumc
        prev_eq = eq

    def lane_reduce(v):
        v2 = v[:, :64] + v[:, 64:]
        return v2[:, :N_FEAT] + v2[:, N_FEAT:]

    xk_u = lane_reduce(xk_acc)
    we_u = lane_reduce(we_acc)
    sh32 = lane_reduce(sh_acc)
    yk_u = lane_reduce(yk_acc)
    h_u = lane_reduce(h_acc)
    dk_logit = lane_reduce(dk_acc)
    dk1_logit = lane_reduce(dk1_acc)

    eq63 = prev_eq[:, 96:]
    ps_last = p_ref[:, N_PAR * N_FEAT - N_FEAT:]
    dk1_logit = dk1_logit + ps_last * eq63

    cxn = (xf - x0) / sw32
    cyn = (yf - y0) / sh32
    w = (we_u - xk_u) * cxn
    xk = x0 + xk_u * cxn
    h = h_u * cyn
    yk = y0 + yk_u * cyn
    dk = jax.nn.softplus(dk_logit)
    dk1 = jax.nn.softplus(dk1_logit)

    s = h / w
    eps = (x - xk) / w
    e1me = eps * (1.0 - eps)
    e2 = eps * eps
    den = s + (dk1 + dk - 2.0 * s) * e1me
    y = yk + h * (s * e2 + dk * e1me) / den
    num_J = s * s * (dk1 * e2 + 2.0 * s * e1me + dk * (1.0 - eps) ** 2)
    y_ref[...] = y
    ld_ref[...] = jnp.sum(jnp.log(num_J / (den * den)), axis=1, keepdims=True)


def pltpu_roll(v, n):
    return jnp.roll(v, n, axis=1)


@functools.partial(jax.jit, static_argnames=("interpret",))
def kernel(x, parameters, x0, xf, y0, yf, interpret=False):
    batch = x.shape[0]
    p2d = parameters.reshape(batch, N_PAR * N_FEAT)
    grid = (batch // BLK,)
    y, ld = pl.pallas_call(
        _spline_block_kernel,
        grid=grid,
        in_specs=[
            pl.BlockSpec((BLK, N_FEAT), lambda i: (i, 0)),
            pl.BlockSpec((BLK, N_PAR * N_FEAT), lambda i: (i, 0)),
            pl.BlockSpec((1, N_FEAT), lambda i: (0, 0)),
            pl.BlockSpec((1, N_FEAT), lambda i: (0, 0)),
            pl.BlockSpec((1, N_FEAT), lambda i: (0, 0)),
            pl.BlockSpec((1, N_FEAT), lambda i: (0, 0)),
        ],
        out_specs=[
            pl.BlockSpec((BLK, N_FEAT), lambda i: (i, 0)),
            pl.BlockSpec((BLK, 1), lambda i: (i, 0)),
        ],
        out_shape=[
            jax.ShapeDtypeStruct((batch, N_FEAT), jnp.float32),
            jax.ShapeDtypeStruct((batch, 1), jnp.float32),
        ],
        interpret=interpret,
    )(x, p2d, x0.reshape(1, -1), xf.reshape(1, -1),
      y0.reshape(1, -1), yf.reshape(1, -1))
    return y, ld.reshape(batch)


# speedup vs baseline: 2.8249x; 1.8573x over previous
import functools

import jax
import jax.numpy as jnp
from jax.experimental import pallas as pl

N_FEAT = 32
N_BINS = 64
N_PAR = 3 * N_BINS + 1
BLK = 512
CH = 128
NC = N_BINS * N_FEAT // CH


def _spline_block_kernel(x_ref, p_ref, x0_ref, xf_ref, y0_ref, yf_ref,
                         y_ref, ld_ref):
    f32 = jnp.float32
    x = x_ref[...]
    x0 = x0_ref[...]
    xf = xf_ref[...]
    y0 = y0_ref[...]
    yf = yf_ref[...]
    blk = x.shape[0]

    li = jax.lax.broadcasted_iota(jnp.int32, (1, CH), 1)
    m_ge32 = li >= 32
    m_ge64 = li >= 64
    m_ge96 = li >= 96
    m_lt32 = li < 32
    m_lt96 = li < 96

    def roll(v, n):
        return pltpu_roll(v, n)

    carry = jnp.zeros((blk, CH), f32)
    cums = []
    for kc in range(NC):
        ew = jnp.exp(p_ref[:, kc * CH:(kc + 1) * CH])
        a1 = ew + jnp.where(m_ge32, roll(ew, 32), 0.0)
        a2 = a1 + jnp.where(m_ge64, roll(a1, 64), 0.0)
        cums.append(carry + a2)
        tot = jnp.where(m_ge96, a2, 0.0)
        tot = tot + roll(tot, 32)
        tot = tot + roll(tot, 64)
        carry = carry + tot

    sw32 = carry[:, :N_FEAT]
    tb32 = (x - x0) * sw32 / (xf - x0)
    tb = jnp.concatenate([tb32, tb32, tb32, tb32], axis=1)

    zero = jnp.zeros((blk, CH), f32)
    prev_cum = zero
    prev_eq = zero
    xk_acc = zero
    we_acc = zero
    sh_acc = zero
    yk_acc = zero
    h_acc = zero
    dk_acc = zero
    dk1_acc = zero
    off_h = N_BINS * N_FEAT
    off_s = 2 * N_BINS * N_FEAT
    for kc in range(NC):
        cumc = cums[kc]
        cumprev = jnp.where(m_lt32, roll(prev_cum, 32), roll(cumc, 32))
        lt = jnp.where(tb > cumc, 1.0, 0.0)
        if kc == NC - 1:
            lt = jnp.where(m_lt96, lt, 0.0)
        ltp = jnp.where(tb > cumprev, 1.0, 0.0)
        eq = ltp - lt
        xk_acc = xk_acc + cumprev * eq
        we_acc = we_acc + cumc * eq
        eh = jnp.exp(p_ref[:, off_h + kc * CH:off_h + (kc + 1) * CH])
        sh_acc = sh_acc + eh
        yk_acc = yk_acc + eh * lt
        h_acc = h_acc + eh * eq
        ps_c = p_ref[:, off_s + kc * CH:off_s + (kc + 1) * CH]
        dk_acc = dk_acc + ps_c * eq
        sh_eq = jnp.where(m_lt32, roll(prev_eq, 32), roll(eq, 32))
        dk1_acc = dk1_acc + ps_c * sh_eq
        prev_cum = cumc
        prev_eq = eq

    def lane_reduce(v):
        v2 = v[:, :64] + v[:, 64:]
        return v2[:, :N_FEAT] + v2[:, N_FEAT:]

    xk_u = lane_reduce(xk_acc)
    we_u = lane_reduce(we_acc)
    sh32 = lane_reduce(sh_acc)
    yk_u = lane_reduce(yk_acc)
    h_u = lane_reduce(h_acc)
    dk_logit = lane_reduce(dk_acc)
    dk1_logit = lane_reduce(dk1_acc)

    eq63 = prev_eq[:, 96:]
    ps_last = p_ref[:, N_PAR * N_FEAT - N_FEAT:]
    dk1_logit = dk1_logit + ps_last * eq63

    cxn = (xf - x0) / sw32
    cyn = (yf - y0) / sh32
    w = (we_u - xk_u) * cxn
    xk = x0 + xk_u * cxn
    h = h_u * cyn
    yk = y0 + yk_u * cyn
    dk = jax.nn.softplus(dk_logit)
    dk1 = jax.nn.softplus(dk1_logit)

    s = h / w
    eps = (x - xk) / w
    e1me = eps * (1.0 - eps)
    e2 = eps * eps
    den = s + (dk1 + dk - 2.0 * s) * e1me
    y = yk + h * (s * e2 + dk * e1me) / den
    num_J = s * s * (dk1 * e2 + 2.0 * s * e1me + dk * (1.0 - eps) ** 2)
    y_ref[...] = y
    ld_ref[...] = jnp.sum(jnp.log(num_J / (den * den)), axis=1, keepdims=True)


def pltpu_roll(v, n):
    return jnp.roll(v, n, axis=1)


@functools.partial(jax.jit, static_argnames=("interpret",))
def kernel(x, parameters, x0, xf, y0, yf, interpret=False):
    batch = x.shape[0]
    p2d = parameters.reshape(batch, N_PAR * N_FEAT)
    grid = (batch // BLK,)
    y, ld = pl.pallas_call(
        _spline_block_kernel,
        grid=grid,
        in_specs=[
            pl.BlockSpec((BLK, N_FEAT), lambda i: (i, 0)),
            pl.BlockSpec((BLK, N_PAR * N_FEAT), lambda i: (i, 0)),
            pl.BlockSpec((1, N_FEAT), lambda i: (0, 0)),
            pl.BlockSpec((1, N_FEAT), lambda i: (0, 0)),
            pl.BlockSpec((1, N_FEAT), lambda i: (0, 0)),
            pl.BlockSpec((1, N_FEAT), lambda i: (0, 0)),
        ],
        out_specs=[
            pl.BlockSpec((BLK, N_FEAT), lambda i: (i, 0)),
            pl.BlockSpec((BLK, 1), lambda i: (i, 0)),
        ],
        out_shape=[
            jax.ShapeDtypeStruct((batch, N_FEAT), jnp.float32),
            jax.ShapeDtypeStruct((batch, 1), jnp.float32),
        ],
        interpret=interpret,
    )(x, p2d, x0.reshape(1, -1), xf.reshape(1, -1),
      y0.reshape(1, -1), yf.reshape(1, -1))
    return y, ld.reshape(batch)
